# baseline (speedup 1.0000x reference)
import functools

import jax
import jax.numpy as jnp
from jax import lax
from jax.experimental import pallas as pl
from jax.experimental.pallas import tpu as pltpu
from jax.experimental.pallas import tpu_sc as plsc

VOCAB = 1000000
EMB = 64
BATCH = 4096
HIST = 200

NC = 2
NS = 16
NW = NC * NS
LANE = 128
NBLK = VOCAB // LANE
TAIL = VOCAB - NBLK * LANE
BLK_W = NBLK // NW
WLIN_ROWS = VOCAB * EMB // LANE
TCUT = VOCAB - TAIL


def _transpose_64xk_to_flat(src, dst):
    lanes = jnp.arange(16, dtype=jnp.int32)
    rows = [lanes + 16 * t for t in range(EMB // 16)]
    for k in range(LANE):
        col = k % 2 * EMB
        cols16 = jnp.full((16,), k, jnp.int32)
        for t in range(EMB // 16):
            v = plsc.load_gather(src, [rows[t], cols16])
            dst[k // 2, pl.ds(col + 16 * t, 16)] = v


def _transpose_chunk(src, dst):
    lanes = jnp.arange(16, dtype=jnp.int32)
    rows = [lanes + 16 * t for t in range(LANE // 16)]
    for e in range(EMB):
        cols16 = jnp.full((16,), e, jnp.int32)
        for t in range(LANE // 16):
            v = plsc.load_gather(src, [rows[t], cols16])
            dst[e, pl.ds(16 * t, 16)] = v


def _wprep_kernel(wt_hbm, wlin_hbm, g0, g1, t0, t1, gs0, gs1, os0, os1):
    gbufs, tbufs = (g0, g1), (t0, t1)
    gsems, osems = (gs0, gs1), (os0, os1)
    wid = lax.axis_index("s") * NC + lax.axis_index("c")
    j0 = wid * BLK_W

    def rd(j, s):
        return pltpu.make_async_copy(
            wt_hbm.at[:, pl.ds(j * LANE, LANE)], gbufs[s], gsems[s])

    def wr(j, s):
        return pltpu.make_async_copy(
            tbufs[s], wlin_hbm.at[pl.ds(j * EMB, EMB)], osems[s])

    rd(j0, 0).start()
    rd(j0 + 1, 1).start()

    def body(g, carry):
        for s in range(2):
            j = j0 + 2 * g + s

            rd(j, s).wait()

            @pl.when(g >= 1)
            def _():
                wr(j - 2, s).wait()

            _transpose_64xk_to_flat(gbufs[s], tbufs[s])

            @pl.when(2 * g + s + 2 < BLK_W)
            def _():
                rd(j + 2, s).start()

            wr(j, s).start()
        return carry
    lax.fori_loop(0, BLK_W // 2, body, 0)
    wr(j0 + BLK_W - 2, 0).wait()
    wr(j0 + BLK_W - 1, 1).wait()

    @pl.when(wid < 4)
    def _():
        j = NW * BLK_W + wid
        rd(j, 0).start()
        rd(j, 0).wait()
        _transpose_64xk_to_flat(gbufs[0], tbufs[0])
        wr(j, 0).start()
        wr(j, 0).wait()


def _gather_kernel(xt_hbm, wlin_hbm, wtail_hbm, embi_hbm,
                   idx_vm, wtail_vm, g0, g1, t0, t1, gs0, gs1, os0, os1):
    gbufs, tbufs = (g0, g1), (t0, t1)
    gsems, osems = (gs0, gs1), (os0, os1)
    wid = lax.axis_index("s") * NC + lax.axis_index("c")

    pltpu.sync_copy(xt_hbm.at[:, pl.ds(wid * LANE, LANE)], idx_vm)
    pltpu.sync_copy(wtail_hbm, wtail_vm)

    def gather(h, s):
        return pltpu.make_async_copy(
            wlin_hbm.at[idx_vm.at[h]], gbufs[s], gsems[s])

    def wr(h, s):
        return pltpu.make_async_copy(
            tbufs[s], embi_hbm.at[h, wid], osems[s])

    gather(0, 0).start()
    gather(1, 1).start()

    def body(g, carry):
        for s in range(2):
            h = 2 * g + s

            gather(h, s).wait()

            @pl.when(g >= 1)
            def _():
                wr(h - 2, s).wait()

            _transpose_chunk(gbufs[s], tbufs[s])

            cnt = jnp.int32(0)
            for t in range(LANE // 16):
                idx16 = idx_vm[h, pl.ds(16 * t, 16)]
                cnt = cnt + jnp.sum(
                    (idx16 >= TCUT).astype(jnp.int32))

            @pl.when(cnt > 0)
            def _():
                for t in range(LANE // 16):
                    idx16 = idx_vm[h, pl.ds(16 * t, 16)]
                    m = idx16 >= TCUT
                    j16 = jnp.where(m, idx16 - TCUT, 0)

                    def fix_e(e, c):
                        e16 = jnp.full((16,), e, jnp.int32)
                        tv = plsc.load_gather(wtail_vm, [j16, e16])
                        cur = tbufs[s][e, pl.ds(16 * t, 16)]
                        tbufs[s][e, pl.ds(16 * t, 16)] = jnp.where(m, tv, cur)
                        return c
                    lax.fori_loop(0, EMB, fix_e, 0)

            @pl.when(2 * g + s + 2 < HIST)
            def _():
                gather(h + 2, s).start()

            wr(h, s).start()
        return carry
    lax.fori_loop(0, HIST // 2, body, 0)
    wr(HIST - 2, 0).wait()
    wr(HIST - 1, 1).wait()


def _outfmt_kernel(embi_hbm, xt_hbm, embt_hbm, maskt_hbm,
                   x_vm, m_vm, v0, v1, v2, v3, r0, r1, r2, r3):
    vbufs = (v0, v1, v2, v3)
    rsems = (r0, r1, r2, r3)
    wid = lax.axis_index("s") * NC + lax.axis_index("c")

    def rd(h, s):
        return pltpu.make_async_copy(embi_hbm.at[h, wid], vbufs[s], rsems[s])

    for s in range(4):
        rd(s, s).start()

    pltpu.sync_copy(xt_hbm.at[:, pl.ds(wid * LANE, LANE)], x_vm)

    def mask_row(h, carry):
        for t in range(LANE // 16):
            v = x_vm[h, pl.ds(16 * t, 16)]
            m_vm[h, pl.ds(16 * t, 16)] = jnp.where(
                v != 0, jnp.float32(1.0), jnp.float32(0.0))
        return carry
    lax.fori_loop(0, HIST, mask_row, 0)
    pltpu.sync_copy(m_vm, maskt_hbm.at[:, pl.ds(wid * LANE, LANE)])

    def body(g, carry):
        for s in range(4):
            h = 4 * g + s
            rd(h, s).wait()
            pltpu.sync_copy(vbufs[s],
                            embt_hbm.at[h, :, pl.ds(wid * LANE, LANE)])

            @pl.when(4 * g + s + 4 < HIST)
            def _():
                rd(h + 4, s).start()
        return carry
    lax.fori_loop(0, HIST // 4, body, 0)


def kernel(x, W):
    xi = x.astype(jnp.int32)
    mesh = plsc.VectorSubcoreMesh(core_axis_name="c", subcore_axis_name="s")

    wprep = functools.partial(
        pl.kernel,
        out_type=[jax.ShapeDtypeStruct((WLIN_ROWS, LANE), jnp.float32)],
        mesh=mesh,
        compiler_params=pltpu.CompilerParams(use_tc_tiling_on_sc=True, needs_layout_passes=False),
        scratch_types=[
            pltpu.VMEM((EMB, LANE), jnp.float32),
            pltpu.VMEM((EMB, LANE), jnp.float32),
            pltpu.VMEM((EMB, LANE), jnp.float32),
            pltpu.VMEM((EMB, LANE), jnp.float32),
            pltpu.SemaphoreType.DMA,
            pltpu.SemaphoreType.DMA,
            pltpu.SemaphoreType.DMA,
            pltpu.SemaphoreType.DMA,
        ],
    )(_wprep_kernel)
    wlin = wprep(W.T)[0]

    gather = functools.partial(
        pl.kernel,
        out_type=[jax.ShapeDtypeStruct((HIST, NW, EMB, LANE), jnp.float32)],
        mesh=mesh,
        compiler_params=pltpu.CompilerParams(
            use_tc_tiling_on_sc=False, needs_layout_passes=False),
        scratch_types=[
            pltpu.VMEM((HIST, LANE), jnp.int32),
            pltpu.VMEM((TAIL, EMB), jnp.float32),
            pltpu.VMEM((LANE, EMB), jnp.float32),
            pltpu.VMEM((LANE, EMB), jnp.float32),
            pltpu.VMEM((EMB, LANE), jnp.float32),
            pltpu.VMEM((EMB, LANE), jnp.float32),
            pltpu.SemaphoreType.DMA,
            pltpu.SemaphoreType.DMA,
            pltpu.SemaphoreType.DMA,
            pltpu.SemaphoreType.DMA,
        ],
    )(_gather_kernel)
    embi = gather(xi.T, wlin.reshape(VOCAB, EMB), W[TCUT:, :])[0]

    outfmt = functools.partial(
        pl.kernel,
        out_type=[
            jax.ShapeDtypeStruct((HIST, EMB, BATCH), jnp.float32),
            jax.ShapeDtypeStruct((HIST, BATCH), jnp.float32),
        ],
        mesh=mesh,
        compiler_params=pltpu.CompilerParams(use_tc_tiling_on_sc=True, needs_layout_passes=False),
        scratch_types=[
            pltpu.VMEM((HIST, LANE), jnp.int32),
            pltpu.VMEM((HIST, LANE), jnp.float32),
            pltpu.VMEM((EMB, LANE), jnp.float32),
            pltpu.VMEM((EMB, LANE), jnp.float32),
            pltpu.VMEM((EMB, LANE), jnp.float32),
            pltpu.VMEM((EMB, LANE), jnp.float32),
            pltpu.SemaphoreType.DMA,
            pltpu.SemaphoreType.DMA,
            pltpu.SemaphoreType.DMA,
            pltpu.SemaphoreType.DMA,
        ],
    )(_outfmt_kernel)
    embt, maskt = outfmt(embi, xi.T)

    return (jnp.transpose(embt, (2, 0, 1)), maskt.T)


# speedup vs baseline: 1.9353x; 1.9353x over previous
import functools

import jax
import jax.numpy as jnp
from jax import lax
from jax.experimental import pallas as pl
from jax.experimental.pallas import tpu as pltpu
from jax.experimental.pallas import tpu_sc as plsc

VOCAB = 1000000
EMB = 64
BATCH = 4096
HIST = 200

NC = 2
NS = 16
NW = NC * NS
LANE = 128
NBLK = VOCAB // LANE
TAIL = VOCAB - NBLK * LANE
BLK_W = NBLK // NW
WLIN_ROWS = VOCAB * EMB // LANE
TCUT = VOCAB - TAIL


def _transpose_64xk_to_flat(src, dst):
    lanes = jnp.arange(16, dtype=jnp.int32)
    rows = [lanes + 16 * t for t in range(EMB // 16)]

    @plsc.parallel_loop(0, LANE, unroll=8)
    def _(k):
        col = k % 2 * EMB
        cols16 = jnp.full((16,), k, jnp.int32)
        for t in range(EMB // 16):
            v = plsc.load_gather(src, [rows[t], cols16])
            dst[k // 2, pl.ds(col + 16 * t, 16)] = v


def _transpose_chunk(src, dst):
    lanes = jnp.arange(16, dtype=jnp.int32)
    rows = [lanes + 16 * t for t in range(LANE // 16)]

    @plsc.parallel_loop(0, EMB, unroll=8)
    def _(e):
        cols16 = jnp.full((16,), e, jnp.int32)
        for t in range(LANE // 16):
            v = plsc.load_gather(src, [rows[t], cols16])
            dst[e, pl.ds(16 * t, 16)] = v


def _wprep_kernel(wt_hbm, wlin_hbm, g0, g1, t0, t1, gs0, gs1, os0, os1):
    gbufs, tbufs = (g0, g1), (t0, t1)
    gsems, osems = (gs0, gs1), (os0, os1)
    wid = lax.axis_index("s") * NC + lax.axis_index("c")
    j0 = wid * BLK_W

    def rd(j, s):
        return pltpu.make_async_copy(
            wt_hbm.at[:, pl.ds(j * LANE, LANE)], gbufs[s], gsems[s])

    def wr(j, s):
        return pltpu.make_async_copy(
            tbufs[s], wlin_hbm.at[pl.ds(j * EMB, EMB)], osems[s])

    rd(j0, 0).start()
    rd(j0 + 1, 1).start()

    def body(g, carry):
        for s in range(2):
            j = j0 + 2 * g + s

            rd(j, s).wait()

            @pl.when(g >= 1)
            def _():
                wr(j - 2, s).wait()

            _transpose_64xk_to_flat(gbufs[s], tbufs[s])

            @pl.when(2 * g + s + 2 < BLK_W)
            def _():
                rd(j + 2, s).start()

            wr(j, s).start()
        return carry
    lax.fori_loop(0, BLK_W // 2, body, 0)
    wr(j0 + BLK_W - 2, 0).wait()
    wr(j0 + BLK_W - 1, 1).wait()

    @pl.when(wid < 4)
    def _():
        j = NW * BLK_W + wid
        rd(j, 0).start()
        rd(j, 0).wait()
        _transpose_64xk_to_flat(gbufs[0], tbufs[0])
        wr(j, 0).start()
        wr(j, 0).wait()


def _gather_kernel(xt_hbm, wlin_hbm, wtail_hbm, embi_hbm,
                   idx_vm, wtail_vm, g0, g1, t0, t1, gs0, gs1, os0, os1):
    gbufs, tbufs = (g0, g1), (t0, t1)
    gsems, osems = (gs0, gs1), (os0, os1)
    wid = lax.axis_index("s") * NC + lax.axis_index("c")

    pltpu.sync_copy(xt_hbm.at[:, pl.ds(wid * LANE, LANE)], idx_vm)
    pltpu.sync_copy(wtail_hbm, wtail_vm)

    def gather(h, s):
        return pltpu.make_async_copy(
            wlin_hbm.at[idx_vm.at[h]], gbufs[s], gsems[s])

    def wr(h, s):
        return pltpu.make_async_copy(
            tbufs[s], embi_hbm.at[h, wid], osems[s])

    gather(0, 0).start()
    gather(1, 1).start()

    def body(g, carry):
        for s in range(2):
            h = 2 * g + s

            gather(h, s).wait()

            @pl.when(g >= 1)
            def _():
                wr(h - 2, s).wait()

            _transpose_chunk(gbufs[s], tbufs[s])

            cnt = jnp.int32(0)
            for t in range(LANE // 16):
                idx16 = idx_vm[h, pl.ds(16 * t, 16)]
                cnt = cnt + jnp.sum(
                    (idx16 >= TCUT).astype(jnp.int32))

            @pl.when(cnt > 0)
            def _():
                for t in range(LANE // 16):
                    idx16 = idx_vm[h, pl.ds(16 * t, 16)]
                    m = idx16 >= TCUT
                    j16 = jnp.where(m, idx16 - TCUT, 0)

                    def fix_e(e, c):
                        e16 = jnp.full((16,), e, jnp.int32)
                        tv = plsc.load_gather(wtail_vm, [j16, e16])
                        cur = tbufs[s][e, pl.ds(16 * t, 16)]
                        tbufs[s][e, pl.ds(16 * t, 16)] = jnp.where(m, tv, cur)
                        return c
                    lax.fori_loop(0, EMB, fix_e, 0)

            @pl.when(2 * g + s + 2 < HIST)
            def _():
                gather(h + 2, s).start()

            wr(h, s).start()
        return carry
    lax.fori_loop(0, HIST // 2, body, 0)
    wr(HIST - 2, 0).wait()
    wr(HIST - 1, 1).wait()


def _outfmt_kernel(embi_hbm, xt_hbm, embt_hbm, maskt_hbm,
                   x_vm, m_vm, v0, v1, v2, v3, r0, r1, r2, r3):
    vbufs = (v0, v1, v2, v3)
    rsems = (r0, r1, r2, r3)
    wid = lax.axis_index("s") * NC + lax.axis_index("c")

    def rd(h, s):
        return pltpu.make_async_copy(embi_hbm.at[h, wid], vbufs[s], rsems[s])

    for s in range(4):
        rd(s, s).start()

    pltpu.sync_copy(xt_hbm.at[:, pl.ds(wid * LANE, LANE)], x_vm)

    def mask_row(h, carry):
        for t in range(LANE // 16):
            v = x_vm[h, pl.ds(16 * t, 16)]
            m_vm[h, pl.ds(16 * t, 16)] = jnp.where(
                v != 0, jnp.float32(1.0), jnp.float32(0.0))
        return carry
    lax.fori_loop(0, HIST, mask_row, 0)
    pltpu.sync_copy(m_vm, maskt_hbm.at[:, pl.ds(wid * LANE, LANE)])

    def body(g, carry):
        for s in range(4):
            h = 4 * g + s
            rd(h, s).wait()
            pltpu.sync_copy(vbufs[s],
                            embt_hbm.at[h, :, pl.ds(wid * LANE, LANE)])

            @pl.when(4 * g + s + 4 < HIST)
            def _():
                rd(h + 4, s).start()
        return carry
    lax.fori_loop(0, HIST // 4, body, 0)


def kernel(x, W):
    xi = x.astype(jnp.int32)
    mesh = plsc.VectorSubcoreMesh(core_axis_name="c", subcore_axis_name="s")

    wprep = functools.partial(
        pl.kernel,
        out_type=[jax.ShapeDtypeStruct((WLIN_ROWS, LANE), jnp.float32)],
        mesh=mesh,
        compiler_params=pltpu.CompilerParams(use_tc_tiling_on_sc=True, needs_layout_passes=False),
        scratch_types=[
            pltpu.VMEM((EMB, LANE), jnp.float32),
            pltpu.VMEM((EMB, LANE), jnp.float32),
            pltpu.VMEM((EMB, LANE), jnp.float32),
            pltpu.VMEM((EMB, LANE), jnp.float32),
            pltpu.SemaphoreType.DMA,
            pltpu.SemaphoreType.DMA,
            pltpu.SemaphoreType.DMA,
            pltpu.SemaphoreType.DMA,
        ],
    )(_wprep_kernel)
    wlin = wprep(W.T)[0]

    gather = functools.partial(
        pl.kernel,
        out_type=[jax.ShapeDtypeStruct((HIST, NW, EMB, LANE), jnp.float32)],
        mesh=mesh,
        compiler_params=pltpu.CompilerParams(
            use_tc_tiling_on_sc=False, needs_layout_passes=False),
        scratch_types=[
            pltpu.VMEM((HIST, LANE), jnp.int32),
            pltpu.VMEM((TAIL, EMB), jnp.float32),
            pltpu.VMEM((LANE, EMB), jnp.float32),
            pltpu.VMEM((LANE, EMB), jnp.float32),
            pltpu.VMEM((EMB, LANE), jnp.float32),
            pltpu.VMEM((EMB, LANE), jnp.float32),
            pltpu.SemaphoreType.DMA,
            pltpu.SemaphoreType.DMA,
            pltpu.SemaphoreType.DMA,
            pltpu.SemaphoreType.DMA,
        ],
    )(_gather_kernel)
    embi = gather(xi.T, wlin.reshape(VOCAB, EMB), W[TCUT:, :])[0]

    outfmt = functools.partial(
        pl.kernel,
        out_type=[
            jax.ShapeDtypeStruct((HIST, EMB, BATCH), jnp.float32),
            jax.ShapeDtypeStruct((HIST, BATCH), jnp.float32),
        ],
        mesh=mesh,
        compiler_params=pltpu.CompilerParams(use_tc_tiling_on_sc=True, needs_layout_passes=False),
        scratch_types=[
            pltpu.VMEM((HIST, LANE), jnp.int32),
            pltpu.VMEM((HIST, LANE), jnp.float32),
            pltpu.VMEM((EMB, LANE), jnp.float32),
            pltpu.VMEM((EMB, LANE), jnp.float32),
            pltpu.VMEM((EMB, LANE), jnp.float32),
            pltpu.VMEM((EMB, LANE), jnp.float32),
            pltpu.SemaphoreType.DMA,
            pltpu.SemaphoreType.DMA,
            pltpu.SemaphoreType.DMA,
            pltpu.SemaphoreType.DMA,
        ],
    )(_outfmt_kernel)
    embt, maskt = outfmt(embi, xi.T)

    return (jnp.transpose(embt, (2, 0, 1)), maskt.T)


# speedup vs baseline: 2.6596x; 1.3743x over previous
import functools

import jax
import jax.numpy as jnp
from jax import lax
from jax.experimental import pallas as pl
from jax.experimental.pallas import tpu as pltpu
from jax.experimental.pallas import tpu_sc as plsc

VOCAB = 1000000
EMB = 64
BATCH = 4096
HIST = 200

NC = 2
NS = 16
NW = NC * NS
LANE = 128
NBLK = VOCAB // LANE
TAIL = VOCAB - NBLK * LANE
BLK_W = NBLK // NW
WLIN_ROWS = VOCAB * EMB // LANE
TCUT = VOCAB - TAIL


def _transpose_64xk_to_flat(src, dst):
    lanes = jnp.arange(16, dtype=jnp.int32)
    kbase = [(lanes + 16 * t) * EMB for t in range(LANE // 16)]

    @plsc.parallel_loop(0, EMB, unroll=8)
    def _(e):
        for t in range(LANE // 16):
            v = src[e, pl.ds(16 * t, 16)]
            f16 = kbase[t] + e
            plsc.store_scatter(dst, [f16 >> 7, f16 & (LANE - 1)], v)


def _transpose_chunk(src, dst):
    lanes = jnp.arange(16, dtype=jnp.int32)
    erows = [lanes + 16 * t for t in range(EMB // 16)]

    @plsc.parallel_loop(0, LANE, unroll=8)
    def _(k):
        cols16 = jnp.full((16,), k, jnp.int32)
        for t in range(EMB // 16):
            v = src[k, pl.ds(16 * t, 16)]
            plsc.store_scatter(dst, [erows[t], cols16], v)


def _wprep_kernel(wt_hbm, wlin_hbm, g0, g1, t0, t1, gs0, gs1, os0, os1):
    gbufs, tbufs = (g0, g1), (t0, t1)
    gsems, osems = (gs0, gs1), (os0, os1)
    wid = lax.axis_index("s") * NC + lax.axis_index("c")
    j0 = wid * BLK_W

    def rd(j, s):
        return pltpu.make_async_copy(
            wt_hbm.at[:, pl.ds(j * LANE, LANE)], gbufs[s], gsems[s])

    def wr(j, s):
        return pltpu.make_async_copy(
            tbufs[s].at[:, pl.ds(0, LANE)],
            wlin_hbm.at[pl.ds(j * EMB, EMB)], osems[s])

    rd(j0, 0).start()
    rd(j0 + 1, 1).start()

    def body(g, carry):
        for s in range(2):
            j = j0 + 2 * g + s

            rd(j, s).wait()

            @pl.when(g >= 1)
            def _():
                wr(j - 2, s).wait()

            _transpose_64xk_to_flat(gbufs[s], tbufs[s])

            @pl.when(2 * g + s + 2 < BLK_W)
            def _():
                rd(j + 2, s).start()

            wr(j, s).start()
        return carry
    lax.fori_loop(0, BLK_W // 2, body, 0)
    wr(j0 + BLK_W - 2, 0).wait()
    wr(j0 + BLK_W - 1, 1).wait()

    @pl.when(wid < 4)
    def _():
        j = NW * BLK_W + wid
        rd(j, 0).start()
        rd(j, 0).wait()
        _transpose_64xk_to_flat(gbufs[0], tbufs[0])
        wr(j, 0).start()
        wr(j, 0).wait()


def _gather_kernel(xt_hbm, wlin_hbm, wtail_hbm, embi_hbm,
                   idx_vm, wtail_vm, g0, g1, t0, t1, gs0, gs1, os0, os1):
    gbufs, tbufs = (g0, g1), (t0, t1)
    gsems, osems = (gs0, gs1), (os0, os1)
    wid = lax.axis_index("s") * NC + lax.axis_index("c")

    pltpu.sync_copy(xt_hbm.at[:, pl.ds(wid * LANE, LANE)], idx_vm)
    pltpu.sync_copy(wtail_hbm, wtail_vm)

    def gather(h, s):
        return pltpu.make_async_copy(
            wlin_hbm.at[idx_vm.at[h]], gbufs[s], gsems[s])

    def wr(h, s):
        return pltpu.make_async_copy(
            tbufs[s].at[:, pl.ds(0, LANE)], embi_hbm.at[h, wid], osems[s])

    gather(0, 0).start()
    gather(1, 1).start()

    def body(g, carry):
        for s in range(2):
            h = 2 * g + s

            gather(h, s).wait()

            @pl.when(g >= 1)
            def _():
                wr(h - 2, s).wait()

            _transpose_chunk(gbufs[s], tbufs[s])

            cnt = jnp.int32(0)
            for t in range(LANE // 16):
                idx16 = idx_vm[h, pl.ds(16 * t, 16)]
                cnt = cnt + jnp.sum(
                    (idx16 >= TCUT).astype(jnp.int32))

            @pl.when(cnt > 0)
            def _():
                for t in range(LANE // 16):
                    idx16 = idx_vm[h, pl.ds(16 * t, 16)]
                    m = idx16 >= TCUT
                    j16 = jnp.where(m, idx16 - TCUT, 0)

                    k16 = jnp.arange(16, dtype=jnp.int32) + 16 * t

                    def fix_e(e, c):
                        e16 = jnp.full((16,), e, jnp.int32)
                        tv = plsc.load_gather(wtail_vm, [j16, e16])
                        plsc.store_scatter(tbufs[s], [e16, k16], tv, mask=m)
                        return c
                    lax.fori_loop(0, EMB, fix_e, 0)

            @pl.when(2 * g + s + 2 < HIST)
            def _():
                gather(h + 2, s).start()

            wr(h, s).start()
        return carry
    lax.fori_loop(0, HIST // 2, body, 0)
    wr(HIST - 2, 0).wait()
    wr(HIST - 1, 1).wait()


def _outfmt_kernel(embi_hbm, xt_hbm, embt_hbm, maskt_hbm,
                   x_vm, m_vm, v0, v1, v2, v3, r0, r1, r2, r3):
    vbufs = (v0, v1, v2, v3)
    rsems = (r0, r1, r2, r3)
    wid = lax.axis_index("s") * NC + lax.axis_index("c")

    def rd(h, s):
        return pltpu.make_async_copy(embi_hbm.at[h, wid], vbufs[s], rsems[s])

    for s in range(4):
        rd(s, s).start()

    pltpu.sync_copy(xt_hbm.at[:, pl.ds(wid * LANE, LANE)], x_vm)

    def mask_row(h, carry):
        for t in range(LANE // 16):
            v = x_vm[h, pl.ds(16 * t, 16)]
            m_vm[h, pl.ds(16 * t, 16)] = jnp.where(
                v != 0, jnp.float32(1.0), jnp.float32(0.0))
        return carry
    lax.fori_loop(0, HIST, mask_row, 0)
    pltpu.sync_copy(m_vm, maskt_hbm.at[:, pl.ds(wid * LANE, LANE)])

    def body(g, carry):
        for s in range(4):
            h = 4 * g + s
            rd(h, s).wait()
            pltpu.sync_copy(vbufs[s],
                            embt_hbm.at[h, :, pl.ds(wid * LANE, LANE)])

            @pl.when(4 * g + s + 4 < HIST)
            def _():
                rd(h + 4, s).start()
        return carry
    lax.fori_loop(0, HIST // 4, body, 0)


def kernel(x, W):
    xi = x.astype(jnp.int32)
    mesh = plsc.VectorSubcoreMesh(core_axis_name="c", subcore_axis_name="s")

    wprep = functools.partial(
        pl.kernel,
        out_type=[jax.ShapeDtypeStruct((WLIN_ROWS, LANE), jnp.float32)],
        mesh=mesh,
        compiler_params=pltpu.CompilerParams(use_tc_tiling_on_sc=True, needs_layout_passes=False),
        scratch_types=[
            pltpu.VMEM((EMB, LANE), jnp.float32),
            pltpu.VMEM((EMB, LANE), jnp.float32),
            pltpu.VMEM((EMB, LANE + 1), jnp.float32),
            pltpu.VMEM((EMB, LANE + 1), jnp.float32),
            pltpu.SemaphoreType.DMA,
            pltpu.SemaphoreType.DMA,
            pltpu.SemaphoreType.DMA,
            pltpu.SemaphoreType.DMA,
        ],
    )(_wprep_kernel)
    wlin = wprep(W.T)[0]

    gather = functools.partial(
        pl.kernel,
        out_type=[jax.ShapeDtypeStruct((HIST, NW, EMB, LANE), jnp.float32)],
        mesh=mesh,
        compiler_params=pltpu.CompilerParams(
            use_tc_tiling_on_sc=False, needs_layout_passes=False),
        scratch_types=[
            pltpu.VMEM((HIST, LANE), jnp.int32),
            pltpu.VMEM((TAIL, EMB), jnp.float32),
            pltpu.VMEM((LANE, EMB), jnp.float32),
            pltpu.VMEM((LANE, EMB), jnp.float32),
            pltpu.VMEM((EMB, LANE + 1), jnp.float32),
            pltpu.VMEM((EMB, LANE + 1), jnp.float32),
            pltpu.SemaphoreType.DMA,
            pltpu.SemaphoreType.DMA,
            pltpu.SemaphoreType.DMA,
            pltpu.SemaphoreType.DMA,
        ],
    )(_gather_kernel)
    embi = gather(xi.T, wlin.reshape(VOCAB, EMB), W[TCUT:, :])[0]

    outfmt = functools.partial(
        pl.kernel,
        out_type=[
            jax.ShapeDtypeStruct((HIST, EMB, BATCH), jnp.float32),
            jax.ShapeDtypeStruct((HIST, BATCH), jnp.float32),
        ],
        mesh=mesh,
        compiler_params=pltpu.CompilerParams(use_tc_tiling_on_sc=True, needs_layout_passes=False),
        scratch_types=[
            pltpu.VMEM((HIST, LANE), jnp.int32),
            pltpu.VMEM((HIST, LANE), jnp.float32),
            pltpu.VMEM((EMB, LANE), jnp.float32),
            pltpu.VMEM((EMB, LANE), jnp.float32),
            pltpu.VMEM((EMB, LANE), jnp.float32),
            pltpu.VMEM((EMB, LANE), jnp.float32),
            pltpu.SemaphoreType.DMA,
            pltpu.SemaphoreType.DMA,
            pltpu.SemaphoreType.DMA,
            pltpu.SemaphoreType.DMA,
        ],
    )(_outfmt_kernel)
    embt, maskt = outfmt(embi, xi.T)

    return (jnp.transpose(embt, (2, 0, 1)), maskt.T)
